# baseline (speedup 1.0000x reference)
import functools

import jax
import jax.numpy as jnp
from jax import lax
from jax.experimental import pallas as pl
from jax.experimental.pallas import tpu as pltpu
from jax.experimental.pallas import tpu_sc as plsc

NW = 32
CH = 128


def kernel(x, token_table, pos_table):
    B, S = x.shape
    V, D = token_table.shape
    N = B * S
    per_w = N // NW
    nch = per_w // CH
    xf = x.reshape(NW, nch, CH).astype(jnp.int32)
    pair = xf >> 1
    parity = xf & 1
    tok2 = token_table.reshape(V // 2, 2 * D)
    mesh = plsc.VectorSubcoreMesh(core_axis_name="c", subcore_axis_name="s")

    @functools.partial(
        pl.kernel,
        mesh=mesh,
        out_type=jax.ShapeDtypeStruct((N, D), jnp.float32),
        scratch_types=[
            pltpu.VMEM((nch, CH), jnp.int32),
            pltpu.VMEM((nch, CH), jnp.int32),
            pltpu.VMEM((CH, 2 * D), jnp.float32),
            pltpu.VMEM((CH, D), jnp.float32),
            pltpu.SemaphoreType.DMA,
        ],
        compiler_params=pltpu.CompilerParams(
            needs_layout_passes=False, use_tc_tiling_on_sc=True),
    )
    def run(pair_hbm, par_hbm, tok_hbm, pos_hbm, out_hbm,
            pair_v, par_v, buf_v, out_v, sem):
        cid = lax.axis_index("c")
        sid = lax.axis_index("s")
        wid = sid * 2 + cid
        base = wid * per_w
        s_base = lax.rem(base, S)
        pltpu.sync_copy(pair_hbm.at[wid], pair_v)
        pltpu.sync_copy(par_hbm.at[wid], par_v)
        lanes = lax.iota(jnp.int32, 16)

        def chunk(c, carry):
            g = pltpu.async_copy(tok_hbm.at[pair_v.at[c]], buf_v, sem)
            pltpu.sync_copy(pos_hbm.at[pl.ds(s_base + c * CH, CH)], out_v)
            g.wait()

            def group(g2, carry2):
                ri = g2 * 16 + lanes
                pv = par_v[c, pl.ds(g2 * 16, 16)] * D

                def col(c2, carry3):
                    c2v = jnp.full((16,), 0, jnp.int32) + c2
                    v = plsc.load_gather(buf_v, [ri, pv + c2v])
                    plsc.addupdate_scatter(out_v, [ri, c2v], v)
                    return carry3

                lax.fori_loop(0, D, col, 0)
                return carry2

            lax.fori_loop(0, CH // 16, group, 0)
            pltpu.sync_copy(out_v, out_hbm.at[pl.ds(base + c * CH, CH)])
            return carry

        lax.fori_loop(0, nch, chunk, 0)

    out = run(pair, parity, tok2, pos_table)
    return out.reshape(B, S, D)


# speedup vs baseline: 1.1192x; 1.1192x over previous
import functools

import jax
import jax.numpy as jnp
from jax import lax
from jax.experimental import pallas as pl
from jax.experimental.pallas import tpu as pltpu
from jax.experimental.pallas import tpu_sc as plsc

NW = 32
CH = 128


def kernel(x, token_table, pos_table):
    B, S = x.shape
    V, D = token_table.shape
    N = B * S
    per_w = N // NW
    nch = per_w // CH
    idx = x.reshape(NW, nch, CH).astype(jnp.int32)
    mesh = plsc.VectorSubcoreMesh(core_axis_name="c", subcore_axis_name="s")

    @functools.partial(
        pl.kernel,
        mesh=mesh,
        out_type=jax.ShapeDtypeStruct((N, D), jnp.float32),
        scratch_types=[
            pltpu.VMEM((nch, CH), jnp.int32),
            pltpu.VMEM((CH, D), jnp.float32),
            pltpu.VMEM((CH, D), jnp.float32),
            pltpu.SemaphoreType.DMA,
            pltpu.SemaphoreType.DMA,
        ],
        compiler_params=pltpu.CompilerParams(use_tc_tiling_on_sc=False),
    )
    def run(x_hbm, tok_hbm, pos_hbm, out_hbm, idx_v, tok_v, out_v, gsem, psem):
        cid = lax.axis_index("c")
        sid = lax.axis_index("s")
        wid = sid * 2 + cid
        base = wid * per_w
        s_base = lax.rem(base, S)
        pltpu.sync_copy(x_hbm.at[wid], idx_v)

        def chunk(c, carry):
            g = pltpu.async_copy(tok_hbm.at[idx_v.at[c]], tok_v, gsem)
            pltpu.sync_copy(pos_hbm.at[pl.ds(s_base + c * CH, CH)], out_v)
            g.wait()

            def row(i, carry2):
                for j in range(D // 16):
                    sl = pl.ds(j * 16, 16)
                    out_v[i, sl] = out_v[i, sl] + tok_v[i, sl]
                return carry2

            lax.fori_loop(0, CH, row, 0)
            pltpu.sync_copy(out_v, out_hbm.at[pl.ds(base + c * CH, CH)])
            return carry

        lax.fori_loop(0, nch, chunk, 0)

    out = run(idx, token_table, pos_table)
    return out.reshape(B, S, D)
